# Initial kernel scaffold; baseline (speedup 1.0000x reference)
import jax
import jax.numpy as jnp
from jax.experimental import pallas as pl


def kernel(images, theta):
    raise NotImplementedError("write your pallas kernel here")


# speedup vs baseline: 3.0034x; 3.0034x over previous
import functools

import jax
import jax.numpy as jnp
from jax import lax
from jax.experimental import pallas as pl
from jax.experimental.pallas import tpu as pltpu
from jax.experimental.pallas import tpu_sc as plsc

B = 8
H = 512
W = 512
CH = 96
NPIX = B * H * W
NUM_CORES = 2
NUM_SUBCORES = 16
NW = NUM_CORES * NUM_SUBCORES
PIX_PER_W = NPIX // NW
CHUNK = 64
NCHUNKS = PIX_PER_W // CHUNK
STEP = 2.0 / 511.0
LANES = 16


def _rne_bf16(x):
    u = lax.bitcast_convert_type(x, jnp.uint32)
    r = (u + jnp.uint32(32767) + ((u >> 16) & jnp.uint32(1))) & jnp.uint32(0xFFFF0000)
    return lax.bitcast_convert_type(r, jnp.float32)


def _compute_chunk_indices(theta_s, bimg, hrow, w0, xg_v, yg_v,
                           ia_v, ib_v, ic_v, id_v, wts_v):
    t00, t01, t02, t10, t11, t12 = theta_s
    yy = yg_v[hrow, :]
    rb = bimg * (H * W)
    wacc = jnp.zeros((LANES,), jnp.float32)
    for j in range(CHUNK // LANES):
        xx = xg_v[w0 // LANES + j, :]
        xt = t00 * xx + t01 * yy + t02
        yt = t10 * xx + t11 * yy + t12
        xf = (xt + 1.0) * (0.5 * W)
        yf = (yt + 1.0) * (0.5 * H)
        xfc = jnp.clip(xf, -4.0, 4.0 + W)
        yfc = jnp.clip(yf, -4.0, 4.0 + H)
        xi = xfc.astype(jnp.int32)
        yi = yfc.astype(jnp.int32)
        x0 = jnp.where(xi.astype(jnp.float32) > xfc, xi - 1, xi)
        y0 = jnp.where(yi.astype(jnp.float32) > yfc, yi - 1, yi)
        x0c = jnp.clip(x0, 0, W - 1)
        x1c = jnp.clip(x0 + 1, 0, W - 1)
        y0c = jnp.clip(y0, 0, H - 1)
        y1c = jnp.clip(y0 + 1, 0, H - 1)
        xcl = jnp.clip(xfc, 0.0, float(W - 1))
        ycl = jnp.clip(yfc, 0.0, float(H - 1))
        wx0 = x1c.astype(jnp.float32) - xcl
        wx1 = xcl - x0c.astype(jnp.float32)
        wy0 = y1c.astype(jnp.float32) - ycl
        wy1 = ycl - y0c.astype(jnp.float32)
        sl = pl.ds(j * LANES, LANES)
        r0 = rb + y0c * W
        r1 = rb + y1c * W
        ia_v[sl] = r0 + x0c
        ib_v[sl] = r1 + x0c
        ic_v[sl] = r0 + x1c
        id_v[sl] = r1 + x1c
        wa = wx0 * wy0
        wb = wx0 * wy1
        wc = wx1 * wy0
        wd = wx1 * wy1
        wts_v[0, sl] = wa
        wts_v[1, sl] = wb
        wts_v[2, sl] = wc
        wts_v[3, sl] = wd
        wacc = wacc + ((wa + wb) + (wc + wd))
    cnt = plsc.all_reduce_population_count(wacc > 0.0)
    return cnt[0] > 0


@functools.partial(
    pl.kernel,
    mesh=plsc.VectorSubcoreMesh(core_axis_name="c", subcore_axis_name="s"),
    compiler_params=pltpu.CompilerParams(use_tc_tiling_on_sc=False,
                                         needs_layout_passes=False),
    out_type=jax.ShapeDtypeStruct((NPIX, CH), jnp.float32),
    scratch_types=[
        pltpu.VMEM((8, 16), jnp.float32),
        pltpu.VMEM((W // 16, 16), jnp.float32),
        pltpu.VMEM((H, 16), jnp.float32),
        pltpu.VMEM((CHUNK,), jnp.int32),
        pltpu.VMEM((CHUNK,), jnp.int32),
        pltpu.VMEM((CHUNK,), jnp.int32),
        pltpu.VMEM((CHUNK,), jnp.int32),
        pltpu.VMEM((4, CHUNK + 16), jnp.float32),
        pltpu.VMEM((CHUNK, CH), jnp.float32),
        pltpu.VMEM((CHUNK, CH), jnp.float32),
        pltpu.VMEM((CHUNK, CH), jnp.float32),
        pltpu.VMEM((CHUNK, CH), jnp.float32),
        pltpu.VMEM((CHUNK, CH), jnp.float32),
        pltpu.VMEM((CHUNK, CH), jnp.float32),
        pltpu.SemaphoreType.DMA,
    ],
)
def _bilinear_sc(tbl_hbm, theta_hbm, xg_hbm, yg_hbm, out_hbm,
                 theta_v, xg_v, yg_v, ia_v, ib_v, ic_v, id_v, wts_v,
                 ga, gb, gc, gd, obuf, zbuf, gsem):
    wid = lax.axis_index("s") * NUM_CORES + lax.axis_index("c")
    pltpu.sync_copy(theta_hbm, theta_v)
    pltpu.sync_copy(xg_hbm, xg_v)
    pltpu.sync_copy(yg_hbm, yg_v)

    def zero_body(c, _):
        for v in range(CH // LANES):
            zbuf[c, pl.ds(v * LANES, LANES)] = jnp.zeros((LANES,), jnp.float32)
        return 0

    lax.fori_loop(0, CHUNK, zero_body, 0)
    base_pix = wid * PIX_PER_W
    bimg = base_pix // (H * W)
    tv = theta_v[bimg, :]
    theta_s = (tv[0], tv[1], tv[2], tv[3], tv[4], tv[5])

    def chunk_body(k, carry):
        pix0 = base_pix + k * CHUNK
        hrow = (pix0 // W) % H
        w0 = pix0 % W
        nonzero = _compute_chunk_indices(theta_s, bimg, hrow, w0, xg_v, yg_v,
                                         ia_v, ib_v, ic_v, id_v, wts_v)

        @pl.when(nonzero)
        def _():
            pltpu.async_copy(tbl_hbm.at[ia_v], ga, gsem).wait()
            pltpu.async_copy(tbl_hbm.at[ib_v], gb, gsem).wait()
            pltpu.async_copy(tbl_hbm.at[ic_v], gc, gsem).wait()
            pltpu.async_copy(tbl_hbm.at[id_v], gd, gsem).wait()

            def px_body(c, _):
                wa = wts_v[0, pl.ds(c, LANES)][0]
                wb = wts_v[1, pl.ds(c, LANES)][0]
                wc = wts_v[2, pl.ds(c, LANES)][0]
                wd = wts_v[3, pl.ds(c, LANES)][0]
                for v in range(CH // LANES):
                    sl = pl.ds(v * LANES, LANES)
                    obuf[c, sl] = (wa * ga[c, sl] + wb * gb[c, sl]
                                   + wc * gc[c, sl] + wd * gd[c, sl])
                return 0

            lax.fori_loop(0, CHUNK, px_body, 0)
            pltpu.sync_copy(obuf, out_hbm.at[pl.ds(pix0, CHUNK)])

        @pl.when(jnp.logical_not(nonzero))
        def _():
            pltpu.sync_copy(zbuf, out_hbm.at[pl.ds(pix0, CHUNK)])

        return 0

    lax.fori_loop(0, NCHUNKS, chunk_body, 0)


def kernel(images, theta):
    tbl = images.reshape(NPIX, CH)
    th6 = _rne_bf16(theta.reshape(B, 6))
    th = jnp.zeros((B, 16), jnp.float32).at[:, :6].set(th6)
    grid = (jnp.arange(W, dtype=jnp.float32) * jnp.float32(STEP) - 1.0)
    grid = _rne_bf16(grid)
    xg = grid.reshape(W // 16, 16)
    yg = jnp.broadcast_to(grid[:, None], (H, 16))
    out = _bilinear_sc(tbl, th, xg, yg)
    return out.reshape(B, H, W, CH)


# speedup vs baseline: 3.5929x; 1.1963x over previous
import functools

import jax
import jax.numpy as jnp
from jax import lax
from jax.experimental import pallas as pl
from jax.experimental.pallas import tpu as pltpu
from jax.experimental.pallas import tpu_sc as plsc

B = 8
H = 512
W = 512
CH = 96
NPIX = B * H * W
NUM_CORES = 2
NUM_SUBCORES = 16
NW = NUM_CORES * NUM_SUBCORES
PIX_PER_W = NPIX // NW
CHUNK = 64
NCHUNKS = PIX_PER_W // CHUNK
STEP = 2.0 / 511.0
LANES = 16


def _rne_bf16(x):
    u = lax.bitcast_convert_type(x, jnp.uint32)
    r = (u + jnp.uint32(32767) + ((u >> 16) & jnp.uint32(1))) & jnp.uint32(0xFFFF0000)
    return lax.bitcast_convert_type(r, jnp.float32)


def _compute_chunk_indices(theta_s, bimg, hrow, w0, xg_v, yg_v,
                           ia_v, ib_v, ic_v, id_v, wts_v):
    t00, t01, t02, t10, t11, t12 = theta_s
    yy = yg_v[hrow, :]
    rb = bimg * (H * W)
    wacc = jnp.zeros((LANES,), jnp.float32)
    for j in range(CHUNK // LANES):
        xx = xg_v[w0 // LANES + j, :]
        xt = t00 * xx + t01 * yy + t02
        yt = t10 * xx + t11 * yy + t12
        xf = (xt + 1.0) * (0.5 * W)
        yf = (yt + 1.0) * (0.5 * H)
        xfc = jnp.clip(xf, -4.0, 4.0 + W)
        yfc = jnp.clip(yf, -4.0, 4.0 + H)
        xi = xfc.astype(jnp.int32)
        yi = yfc.astype(jnp.int32)
        x0 = jnp.where(xi.astype(jnp.float32) > xfc, xi - 1, xi)
        y0 = jnp.where(yi.astype(jnp.float32) > yfc, yi - 1, yi)
        x0c = jnp.clip(x0, 0, W - 1)
        x1c = jnp.clip(x0 + 1, 0, W - 1)
        y0c = jnp.clip(y0, 0, H - 1)
        y1c = jnp.clip(y0 + 1, 0, H - 1)
        xcl = jnp.clip(xfc, 0.0, float(W - 1))
        ycl = jnp.clip(yfc, 0.0, float(H - 1))
        wx0 = x1c.astype(jnp.float32) - xcl
        wx1 = xcl - x0c.astype(jnp.float32)
        wy0 = y1c.astype(jnp.float32) - ycl
        wy1 = ycl - y0c.astype(jnp.float32)
        sl = pl.ds(j * LANES, LANES)
        r0 = rb + y0c * W
        r1 = rb + y1c * W
        ia_v[sl] = r0 + x0c
        ib_v[sl] = r1 + x0c
        ic_v[sl] = r0 + x1c
        id_v[sl] = r1 + x1c
        wa = wx0 * wy0
        wb = wx0 * wy1
        wc = wx1 * wy0
        wd = wx1 * wy1
        wts_v[0, sl] = wa
        wts_v[1, sl] = wb
        wts_v[2, sl] = wc
        wts_v[3, sl] = wd
        wacc = wacc + ((wa + wb) + (wc + wd))
    cnt = plsc.all_reduce_population_count(wacc > 0.0)
    return cnt[0] > 0


@functools.partial(
    pl.kernel,
    mesh=plsc.VectorSubcoreMesh(core_axis_name="c", subcore_axis_name="s"),
    compiler_params=pltpu.CompilerParams(use_tc_tiling_on_sc=False,
                                         needs_layout_passes=False),
    out_type=jax.ShapeDtypeStruct((NPIX, CH), jnp.float32),
    scratch_types=[
        pltpu.VMEM((8, 16), jnp.float32),
        pltpu.VMEM((W // 16, 16), jnp.float32),
        pltpu.VMEM((H, 16), jnp.float32),
        pltpu.VMEM((CHUNK,), jnp.int32),
        pltpu.VMEM((CHUNK,), jnp.int32),
        pltpu.VMEM((CHUNK,), jnp.int32),
        pltpu.VMEM((CHUNK,), jnp.int32),
        pltpu.VMEM((4, CHUNK + 16), jnp.float32),
        pltpu.VMEM((CHUNK, CH), jnp.float32),
        pltpu.VMEM((CHUNK, CH), jnp.float32),
        pltpu.VMEM((CHUNK, CH), jnp.float32),
        pltpu.VMEM((CHUNK, CH), jnp.float32),
        pltpu.VMEM((CHUNK, CH), jnp.float32),
        pltpu.VMEM((CHUNK, CH), jnp.float32),
        pltpu.SemaphoreType.DMA,
    ],
)
def _bilinear_sc(tbl_hbm, theta_hbm, xg_hbm, yg_hbm, out_hbm,
                 theta_v, xg_v, yg_v, ia_v, ib_v, ic_v, id_v, wts_v,
                 ga, gb, gc, gd, obuf, zbuf, gsem):
    wid = lax.axis_index("s") * NUM_CORES + lax.axis_index("c")
    pltpu.sync_copy(theta_hbm, theta_v)
    pltpu.sync_copy(xg_hbm, xg_v)
    pltpu.sync_copy(yg_hbm, yg_v)

    def zero_body(c, _):
        for v in range(CH // LANES):
            zbuf[c, pl.ds(v * LANES, LANES)] = jnp.zeros((LANES,), jnp.float32)
        return 0

    lax.fori_loop(0, CHUNK, zero_body, 0)
    base_pix = wid * PIX_PER_W
    bimg = base_pix // (H * W)
    tv = theta_v[bimg, :]
    theta_s = (tv[0], tv[1], tv[2], tv[3], tv[4], tv[5])

    def chunk_body(k, carry):
        pix0 = base_pix + k * CHUNK
        hrow = (pix0 // W) % H
        w0 = pix0 % W
        nonzero = _compute_chunk_indices(theta_s, bimg, hrow, w0, xg_v, yg_v,
                                         ia_v, ib_v, ic_v, id_v, wts_v)

        @pl.when(nonzero)
        def _():
            cpa = pltpu.async_copy(tbl_hbm.at[ia_v], ga, gsem)
            cpb = pltpu.async_copy(tbl_hbm.at[ib_v], gb, gsem)
            cpc = pltpu.async_copy(tbl_hbm.at[ic_v], gc, gsem)
            cpd = pltpu.async_copy(tbl_hbm.at[id_v], gd, gsem)
            cpa.wait()
            cpb.wait()
            cpc.wait()
            cpd.wait()

            def px_body(c, _):
                wa = wts_v[0, pl.ds(c, LANES)][0]
                wb = wts_v[1, pl.ds(c, LANES)][0]
                wc = wts_v[2, pl.ds(c, LANES)][0]
                wd = wts_v[3, pl.ds(c, LANES)][0]
                for v in range(CH // LANES):
                    sl = pl.ds(v * LANES, LANES)
                    obuf[c, sl] = (wa * ga[c, sl] + wb * gb[c, sl]
                                   + wc * gc[c, sl] + wd * gd[c, sl])
                return 0

            lax.fori_loop(0, CHUNK, px_body, 0)
            pltpu.sync_copy(obuf, out_hbm.at[pl.ds(pix0, CHUNK)])

        @pl.when(jnp.logical_not(nonzero))
        def _():
            pltpu.sync_copy(zbuf, out_hbm.at[pl.ds(pix0, CHUNK)])

        return 0

    lax.fori_loop(0, NCHUNKS, chunk_body, 0)


def kernel(images, theta):
    tbl = images.reshape(NPIX, CH)
    th6 = _rne_bf16(theta.reshape(B, 6))
    th = jnp.zeros((B, 16), jnp.float32).at[:, :6].set(th6)
    grid = (jnp.arange(W, dtype=jnp.float32) * jnp.float32(STEP) - 1.0)
    grid = _rne_bf16(grid)
    xg = grid.reshape(W // 16, 16)
    yg = jnp.broadcast_to(grid[:, None], (H, 16))
    out = _bilinear_sc(tbl, th, xg, yg)
    return out.reshape(B, H, W, CH)
